# baseline (speedup 1.0000x reference)
import functools
import math

import jax
import jax.numpy as jnp
from jax import lax
from jax.experimental import pallas as pl
from jax.experimental.pallas import tpu as pltpu

B = 2
C = 256
H = 64
W = 64
A = 9
NC = 80
SCORE_THRESH = 0.05
NMS_THRESH = 0.5
TOPK = 1000
DETS = 300
IMG = 512
BBOX_CLAMP = math.log(1000.0 / 16.0)

HW = H * W
NCLS = A * NC
NREG = A * 4

_PREC = lax.Precision.HIGHEST


def _tower_body(x_ref, w_ref, b_ref, out_ref, bufA, bufB, acc):
    bufA[...] = jnp.zeros((H + 2, W + 2, C), jnp.float32)
    bufB[...] = jnp.zeros((H + 2, W + 2, C), jnp.float32)
    out_ref[...] = jnp.zeros((1, H + 2, W + 2, C), jnp.float32)

    def layer(l, src, dst, dst_is_out):
        for mi in range(4):
            h0 = 16 * mi
            first = True
            for dy in range(3):
                for dx in range(3):
                    if src is None:
                        v = x_ref[0, dy + h0:dy + h0 + 16, dx:dx + 64, :]
                    else:
                        v = src[dy + h0:dy + h0 + 16, dx:dx + 64, :]
                    v2 = v.reshape(16 * 64, C)
                    d = jnp.dot(v2, w_ref[l, dy * 3 + dx],
                                preferred_element_type=jnp.float32,
                                precision=_PREC)
                    if first:
                        acc[...] = d
                        first = False
                    else:
                        acc[...] += d
            y = jnp.maximum(acc[...] + b_ref[l], 0.0).reshape(16, 64, C)
            if dst_is_out:
                dst[0, 1 + h0:1 + h0 + 16, 1:65, :] = y
            else:
                dst[1 + h0:1 + h0 + 16, 1:65, :] = y

    layer(0, None, bufA, False)
    layer(1, bufA, bufB, False)
    layer(2, bufB, bufA, False)
    layer(3, bufA, out_ref, True)


def _tower_call(xp, w4, b4):
    return pl.pallas_call(
        _tower_body,
        grid=(B,),
        in_specs=[
            pl.BlockSpec((1, H + 2, W + 2, C), lambda b: (b, 0, 0, 0)),
            pl.BlockSpec((4, 9, C, C), lambda b: (0, 0, 0, 0)),
            pl.BlockSpec((4, 1, C), lambda b: (0, 0, 0)),
        ],
        out_specs=pl.BlockSpec((1, H + 2, W + 2, C), lambda b: (b, 0, 0, 0)),
        out_shape=jax.ShapeDtypeStruct((B, H + 2, W + 2, C), jnp.float32),
        scratch_shapes=[
            pltpu.VMEM((H + 2, W + 2, C), jnp.float32),
            pltpu.VMEM((H + 2, W + 2, C), jnp.float32),
            pltpu.VMEM((16 * 64, C), jnp.float32),
        ],
    )(xp, w4, b4)


def _head_body(nout, sigmoid, x_ref, w_ref, b_ref, out_ref, acc):
    for mi in range(4):
        h0 = 16 * mi
        first = True
        for dy in range(3):
            for dx in range(3):
                v = x_ref[0, dy + h0:dy + h0 + 16, dx:dx + 64, :]
                v2 = v.reshape(16 * 64, C)
                d = jnp.dot(v2, w_ref[dy * 3 + dx],
                            preferred_element_type=jnp.float32,
                            precision=_PREC)
                if first:
                    acc[...] = d
                    first = False
                else:
                    acc[...] += d
        y = acc[...] + b_ref[...]
        if sigmoid:
            y = jax.nn.sigmoid(y)
        out_ref[0, h0 * 64:(h0 + 16) * 64, :] = y


def _head_call(t, wout, bout, nout, sigmoid):
    body = functools.partial(_head_body, nout, sigmoid)
    return pl.pallas_call(
        body,
        grid=(B,),
        in_specs=[
            pl.BlockSpec((1, H + 2, W + 2, C), lambda b: (b, 0, 0, 0)),
            pl.BlockSpec((9, C, nout), lambda b: (0, 0, 0)),
            pl.BlockSpec((1, nout), lambda b: (0, 0)),
        ],
        out_specs=pl.BlockSpec((1, HW, nout), lambda b: (b, 0, 0)),
        out_shape=jax.ShapeDtypeStruct((B, HW, nout), jnp.float32),
        scratch_shapes=[
            pltpu.VMEM((16 * 64, nout), jnp.float32),
        ],
    )(t, wout, bout)


def _decode_boxes(rel, anc):
    widths = anc[:, 2] - anc[:, 0]
    heights = anc[:, 3] - anc[:, 1]
    ctr_x = anc[:, 0] + 0.5 * widths
    ctr_y = anc[:, 1] + 0.5 * heights
    dx, dy, dw, dh = rel[:, 0], rel[:, 1], rel[:, 2], rel[:, 3]
    dw = jnp.minimum(dw, BBOX_CLAMP)
    dh = jnp.minimum(dh, BBOX_CLAMP)
    pcx = dx * widths + ctr_x
    pcy = dy * heights + ctr_y
    pw = jnp.exp(dw) * widths
    ph = jnp.exp(dh) * heights
    return jnp.stack([pcx - 0.5 * pw, pcy - 0.5 * ph,
                      pcx + 0.5 * pw, pcy + 0.5 * ph], axis=1)


def _nms_loop(boxes, valid, thresh):
    n = boxes.shape[0]
    x1, y1, x2, y2 = boxes[:, 0], boxes[:, 1], boxes[:, 2], boxes[:, 3]
    areas = (x2 - x1) * (y2 - y1)
    idx = jnp.arange(n)

    def body(i, carry):
        suppressed, keep = carry
        active = jnp.logical_not(suppressed[i])
        keep = keep.at[i].set(active)
        xx1 = jnp.maximum(x1[i], x1)
        yy1 = jnp.maximum(y1[i], y1)
        xx2 = jnp.minimum(x2[i], x2)
        yy2 = jnp.minimum(y2[i], y2)
        inter = jnp.maximum(0.0, xx2 - xx1) * jnp.maximum(0.0, yy2 - yy1)
        iou = inter / (areas[i] + areas - inter + 1e-12)
        suppressed = suppressed | (active & (idx > i) & (iou > thresh))
        return suppressed, keep

    suppressed0 = jnp.logical_not(valid)
    keep0 = jnp.zeros((n,), dtype=bool)
    _, keep = lax.fori_loop(0, n, body, (suppressed0, keep0))
    return keep


def kernel(features, anchors, cls_conv_w, cls_conv_b, cls_out_w, cls_out_b,
           reg_conv_w, reg_conv_b, reg_out_w, reg_out_b):
    x = jnp.transpose(features, (0, 2, 3, 1))
    xp = jnp.pad(x, ((0, 0), (1, 1), (1, 1), (0, 0)))

    w4c = jnp.transpose(cls_conv_w, (0, 3, 4, 2, 1)).reshape(4, 9, C, C)
    w4r = jnp.transpose(reg_conv_w, (0, 3, 4, 2, 1)).reshape(4, 9, C, C)
    b4c = cls_conv_b.reshape(4, 1, C)
    b4r = reg_conv_b.reshape(4, 1, C)
    wco = jnp.transpose(cls_out_w, (2, 3, 1, 0)).reshape(9, C, NCLS)
    wro = jnp.transpose(reg_out_w, (2, 3, 1, 0)).reshape(9, C, NREG)
    bco = cls_out_b.reshape(1, NCLS)
    bro = reg_out_b.reshape(1, NREG)

    t_cls = _tower_call(xp, w4c, b4c)
    scores = _head_call(t_cls, wco, bco, NCLS, True)
    t_reg = _tower_call(xp, w4r, b4r)
    reg = _head_call(t_reg, wro, bro, NREG, False)

    scores_flat = scores.reshape(B, HW * NCLS)
    reg_flat = reg.reshape(B, HW * A, 4)

    out_scores, out_labels, out_boxes = [], [], []
    for i in range(B):
        sc = scores_flat[i]
        topv, topi = lax.top_k(sc, TOPK)
        valid = topv > SCORE_THRESH
        labels = topi % NC
        aidx = topi // NC
        boxes = _decode_boxes(reg_flat[i][aidx], anchors[aidx])
        boxes = jnp.clip(boxes, 0.0, float(IMG))
        mx = jnp.max(jnp.where(valid[:, None], boxes, -jnp.inf))
        offs = labels.astype(jnp.float32) * (mx + 1.0)
        keep_mask = _nms_loop(boxes + offs[:, None], valid, NMS_THRESH)
        rank = jnp.cumsum(keep_mask) - 1
        dst = jnp.where(keep_mask & (rank < DETS), rank, DETS)
        ps = jnp.full((DETS,), -1.0, jnp.float32).at[dst].set(topv, mode='drop')
        pl_ = jnp.full((DETS,), -1, labels.dtype).at[dst].set(labels, mode='drop')
        pb = jnp.zeros((DETS, 4), jnp.float32).at[dst].set(boxes, mode='drop')
        out_scores.append(ps)
        out_labels.append(pl_)
        out_boxes.append(pb)
    return (jnp.stack(out_scores), jnp.stack(out_labels), jnp.stack(out_boxes))


# speedup vs baseline: 6.1359x; 6.1359x over previous
import functools
import math

import jax
import jax.numpy as jnp
from jax import lax
from jax.experimental import pallas as pl
from jax.experimental.pallas import tpu as pltpu

B = 2
C = 256
H = 64
W = 64
A = 9
NC = 80
SCORE_THRESH = 0.05
NMS_THRESH = 0.5
TOPK = 1000
DETS = 300
IMG = 512
BBOX_CLAMP = math.log(1000.0 / 16.0)

N = 1024
DPAD = 512


def _post_body(topv_ref, lab_in_ref, reg_ref, anc_ref, sco_ref, lab_ref,
               box_ref, m_ref):
    topv = topv_ref[0]
    labels_f = lab_in_ref[0]
    rel = reg_ref[0]
    anc = anc_ref[0]

    valid = topv > SCORE_THRESH

    widths = anc[2:3] - anc[0:1]
    heights = anc[3:4] - anc[1:2]
    ctr_x = anc[0:1] + 0.5 * widths
    ctr_y = anc[1:2] + 0.5 * heights
    dxv, dyv = rel[0:1], rel[1:2]
    dwv = jnp.minimum(rel[2:3], BBOX_CLAMP)
    dhv = jnp.minimum(rel[3:4], BBOX_CLAMP)
    pcx = dxv * widths + ctr_x
    pcy = dyv * heights + ctr_y
    pw = jnp.exp(dwv) * widths
    ph = jnp.exp(dhv) * heights
    x1 = jnp.clip(pcx - 0.5 * pw, 0.0, float(IMG))
    y1 = jnp.clip(pcy - 0.5 * ph, 0.0, float(IMG))
    x2 = jnp.clip(pcx + 0.5 * pw, 0.0, float(IMG))
    y2 = jnp.clip(pcy + 0.5 * ph, 0.0, float(IMG))

    neg_inf = jnp.float32(-jnp.inf)
    mx = jnp.max(jnp.where(valid, jnp.maximum(jnp.maximum(x1, x2),
                                              jnp.maximum(y1, y2)), neg_inf))
    offs = labels_f * (mx + 1.0)
    sx1, sy1, sx2, sy2 = x1 + offs, y1 + offs, x2 + offs, y2 + offs

    c1 = sx1.reshape(N, 1)
    r1 = sx1
    cy1 = sy1.reshape(N, 1)
    ry1 = sy1
    c2 = sx2.reshape(N, 1)
    r2 = sx2
    cy2 = sy2.reshape(N, 1)
    ry2 = sy2
    areas = (sx2 - sx1) * (sy2 - sy1)
    xx1 = jnp.maximum(c1, r1)
    yy1 = jnp.maximum(cy1, ry1)
    xx2 = jnp.minimum(c2, r2)
    yy2 = jnp.minimum(cy2, ry2)
    inter = jnp.maximum(0.0, xx2 - xx1) * jnp.maximum(0.0, yy2 - yy1)
    iou = inter / (areas.reshape(N, 1) + areas - inter + 1e-12)
    ii = lax.broadcasted_iota(jnp.int32, (N, N), 0)
    jj = lax.broadcasted_iota(jnp.int32, (N, N), 1)
    m_ref[...] = ((iou > NMS_THRESH) & (jj > ii)).astype(jnp.float32)

    idx_row = lax.broadcasted_iota(jnp.int32, (1, N), 1)
    supp0 = 1.0 - valid.astype(jnp.float32)

    def body(i, carry):
        supp, keep = carry
        e_i = (idx_row == i).astype(jnp.float32)
        act = 1.0 - jnp.sum(supp * e_i)
        keep = keep + act * e_i
        row = m_ref[pl.ds(i, 1), :]
        supp = jnp.maximum(supp, act * row)
        return supp, keep

    _, keep = lax.fori_loop(0, TOPK, body,
                            (supp0, jnp.zeros((1, N), jnp.float32)))

    lt = (ii <= jj).astype(jnp.float32)
    rank = jnp.dot(keep, lt, preferred_element_type=jnp.float32,
                   precision=lax.Precision.HIGHEST) - 1.0
    slot = lax.broadcasted_iota(jnp.int32, (1, DPAD), 1).astype(jnp.float32)
    onehot = ((rank.reshape(N, 1) == slot) &
              (keep.reshape(N, 1) > 0.5)).astype(jnp.float32)
    hp = lax.Precision.HIGHEST
    ones = jnp.ones((1, N), jnp.float32)
    stk = jnp.concatenate([topv_ref[0], lab_in_ref[0], ones,
                           x1, y1, x2, y2, ones], axis=0)
    res = jnp.dot(stk, onehot, preferred_element_type=jnp.float32,
                  precision=hp)
    filled = res[7:8]
    sco_ref[0] = res[0:1] + filled - 1.0
    lab_ref[0] = res[1:2] + filled - 1.0
    box_ref[0] = res[3:7]


def _post_call(topv, labf, reg_t, anc_t):
    return pl.pallas_call(
        _post_body,
        grid=(B,),
        in_specs=[
            pl.BlockSpec((1, 1, N), lambda b: (b, 0, 0)),
            pl.BlockSpec((1, 1, N), lambda b: (b, 0, 0)),
            pl.BlockSpec((1, 4, N), lambda b: (b, 0, 0)),
            pl.BlockSpec((1, 4, N), lambda b: (b, 0, 0)),
        ],
        out_specs=[
            pl.BlockSpec((1, 1, DPAD), lambda b: (b, 0, 0)),
            pl.BlockSpec((1, 1, DPAD), lambda b: (b, 0, 0)),
            pl.BlockSpec((1, 4, DPAD), lambda b: (b, 0, 0)),
        ],
        out_shape=[
            jax.ShapeDtypeStruct((B, 1, DPAD), jnp.float32),
            jax.ShapeDtypeStruct((B, 1, DPAD), jnp.float32),
            jax.ShapeDtypeStruct((B, 4, DPAD), jnp.float32),
        ],
        scratch_shapes=[pltpu.VMEM((N, N), jnp.float32)],
    )(topv, labf, reg_t, anc_t)


def _conv(x, w, b):
    y = lax.conv_general_dilated(x, w, (1, 1), 'SAME',
                                 dimension_numbers=('NCHW', 'OIHW', 'NCHW'))
    return y + b[None, :, None, None]


def kernel(features, anchors, cls_conv_w, cls_conv_b, cls_out_w, cls_out_b,
           reg_conv_w, reg_conv_b, reg_out_w, reg_out_b):
    t = features
    for i in range(4):
        t = jax.nn.relu(_conv(t, cls_conv_w[i], cls_conv_b[i]))
    logits = _conv(t, cls_out_w, cls_out_b)
    logits = logits.reshape(B, A, NC, H, W).transpose(0, 3, 4, 1, 2)
    t = features
    for i in range(4):
        t = jax.nn.relu(_conv(t, reg_conv_w[i], reg_conv_b[i]))
    reg = _conv(t, reg_out_w, reg_out_b)
    reg = reg.reshape(B, A, 4, H, W).transpose(0, 3, 4, 1, 2).reshape(B, -1, 4)

    scores = jax.nn.sigmoid(logits).reshape(B, -1)

    topv, topi = lax.top_k(scores, TOPK)
    topv = jnp.concatenate(
        [topv, jnp.full((B, N - TOPK), -1.0, jnp.float32)], axis=1)
    topi = jnp.concatenate(
        [topi, jnp.zeros((B, N - TOPK), topi.dtype)], axis=1)
    aidx = topi // NC
    labf = (topi % NC).astype(jnp.float32)
    reg_g = jnp.take_along_axis(reg, aidx[:, :, None], axis=1)
    anc_g = anchors[aidx]
    reg_t = jnp.transpose(reg_g, (0, 2, 1))
    anc_t = jnp.transpose(anc_g, (0, 2, 1))

    sco, lab, box = _post_call(topv.reshape(B, 1, N), labf.reshape(B, 1, N),
                               reg_t, anc_t)

    out_scores = sco[:, 0, :DETS]
    out_labels = lab[:, 0, :DETS].astype(jnp.int32)
    out_boxes = jnp.transpose(box[:, :, :DETS], (0, 2, 1))
    return (out_scores, out_labels, out_boxes)
